# baseline (speedup 1.0000x reference)
import functools

import numpy as np

import jax
import jax.numpy as jnp
from jax import lax
from jax.experimental import pallas as pl
from jax.experimental.pallas import tpu as pltpu
from jax.experimental.pallas import tpu_sc as plsc

_NF = 8
_B = 16384
_L = 20
_V = 1024
_D = 64
_PARTS = 4
_ROWS_PER_W = _B // _PARTS
_CH = 32
_NSUB = _CH * _L // 128
_SCH = 256
_CPS = _SCH // _CH
_IDXROWS = _SCH * _L // 128
_BBM = 1024

_COL_PERM = np.empty(_D, np.int32)
for _k in range(_D // 32):
    for _j in range(16):
        _COL_PERM[_k * 32 + 2 * _j] = _k * 32 + _j
        _COL_PERM[_k * 32 + 2 * _j + 1] = _k * 32 + 16 + _j


def _sc_pool(idx_list, tab, scale):
    mesh = plsc.VectorSubcoreMesh(core_axis_name="c", subcore_axis_name="s")

    @functools.partial(
        pl.kernel,
        mesh=mesh,
        out_type=jax.ShapeDtypeStruct((_NF, _B, _D), jnp.float32),
        scratch_types=[
            pltpu.VMEM((_IDXROWS, 128), jnp.int32),
            pltpu.VMEM((_CH * _L, _D), jnp.int16),
            pltpu.VMEM((_CH * _L, _D), jnp.int16),
            pltpu.VMEM((_CH, _D), jnp.float32),
            pltpu.VMEM((_CH, _D), jnp.float32),
            pltpu.VMEM((16,), jnp.float32),
            pltpu.SemaphoreType.DMA,
            pltpu.SemaphoreType.DMA,
            pltpu.SemaphoreType.DMA,
            pltpu.SemaphoreType.DMA,
        ],
        compiler_params=pltpu.CompilerParams(
            needs_layout_passes=False, use_tc_tiling_on_sc=False
        ),
    )
    def pool(i0, i1, i2, i3, i4, i5, i6, i7, tab_hbm, scale_hbm, out_hbm,
             idx_s, rows0, rows1, outv0, outv1, scale_v,
             sg0, sg1, so0, so1):
        idx_refs = (i0, i1, i2, i3, i4, i5, i6, i7)
        wid = lax.axis_index("s") * 2 + lax.axis_index("c")
        fld = lax.shift_right_logical(wid, 2)
        part = lax.bitwise_and(wid, 3)
        rbase = part * _ROWS_PER_W
        pltpu.sync_copy(scale_hbm, scale_v)
        rows = (rows0, rows1)
        outv = (outv0, outv1)
        sg = (sg0, sg1)
        so = (so0, so1)

        def gathers(k, p):
            for j in range(_NSUB):
                pltpu.make_async_copy(
                    tab_hbm.at[idx_s.at[k * _NSUB + j]],
                    rows[p].at[pl.ds(j * 128, 128)],
                    sg[p],
                ).start()

        def drain_gathers(p):
            pltpu.make_async_copy(
                tab_hbm.at[pl.ds(0, _CH * _L)], rows[p], sg[p]
            ).wait()

        def out_copy(sbase, k, p):
            off = pl.multiple_of(sbase + k * _CH, _CH)
            return pltpu.make_async_copy(
                outv[p],
                out_hbm.at[fld, pl.ds(off, _CH)],
                so[p],
            )

        def compute(p):
            rv = rows[p]
            ov = outv[p]
            s = scale_v[...]

            def tree_sum(vals):
                while len(vals) > 1:
                    nxt = [
                        vals[2 * i] + vals[2 * i + 1]
                        for i in range(len(vals) // 2)
                    ]
                    if len(vals) % 2:
                        nxt.append(vals[-1])
                    vals = nxt
                return vals[0]

            def row_body(r, carry):
                g = r * _L
                for kk in range(_D // 32):
                    sl = pl.ds(kk * 32, 32)
                    evens = []
                    odds = []
                    for l in range(_L):
                        a, b = plsc.unpack(
                            rv[g + l, sl], format=plsc.PackFormat.INTERLEAVED
                        )
                        evens.append(a)
                        odds.append(b)
                    ov[r, pl.ds(kk * 32, 16)] = (
                        tree_sum(evens).astype(jnp.float32) * s
                    )
                    ov[r, pl.ds(kk * 32 + 16, 16)] = (
                        tree_sum(odds).astype(jnp.float32) * s
                    )
                return carry

            lax.fori_loop(0, _CH, row_body, 0)

        vbase = fld * _V

        def superchunk(si, carry):
            sbase = pl.multiple_of(rbase + si * _SCH, _SCH)
            idx_off = pl.multiple_of(sbase * _L // 128, _IDXROWS)
            for f in range(_NF):
                @pl.when(fld == f)
                def _():
                    pltpu.sync_copy(
                        idx_refs[f].at[pl.ds(idx_off, _IDXROWS)], idx_s
                    )

            def rebase_body(q, carry2):
                row = lax.shift_right_logical(q, 3)
                lane = lax.bitwise_and(q, 7) * 16
                sl = pl.ds(lane, 16)
                idx_s[row, sl] = idx_s[row, sl] + vbase
                return carry2

            lax.fori_loop(0, _IDXROWS * 8, rebase_body, 0)

            gathers(0, 0)
            for k in range(_CPS):
                p = k % 2
                if k + 1 < _CPS:
                    gathers(k + 1, 1 - p)
                drain_gathers(p)
                if k >= 2:
                    out_copy(sbase, k - 2, p).wait()
                compute(p)
                out_copy(sbase, k, p).start()
            out_copy(sbase, _CPS - 2, 0).wait()
            out_copy(sbase, _CPS - 1, 1).wait()
            return carry

        lax.fori_loop(0, _ROWS_PER_W // _SCH, superchunk, 0)

    return pool(*idx_list, tab, scale)


def _mlp_body(p_ref, w0_ref, b0_ref, w1_ref, b1_ref, w2_ref, b2_ref,
              wl_ref, bl_ref, out_ref):
    x = jnp.concatenate([p_ref[f] for f in range(_NF)], axis=-1)
    for w_ref, b_ref in ((w0_ref, b0_ref), (w1_ref, b1_ref), (w2_ref, b2_ref)):
        x = jnp.maximum(
            lax.dot(x, w_ref[...], preferred_element_type=jnp.float32)
            + b_ref[...][None, :],
            0.0,
        )
    out_ref[...] = (
        lax.dot(x, wl_ref[...], preferred_element_type=jnp.float32)
        + bl_ref[...][None, :]
    )


def _tc_mlp(pooled, W0, b0, W1, b1, W2, b2, Wl, bl):
    grid = (_B // _BBM,)
    full = lambda shape: pl.BlockSpec(shape, lambda i: tuple(0 for _ in shape))
    in_specs = [
        pl.BlockSpec((_NF, _BBM, _D), lambda i: (0, i, 0)),
        full(W0.shape), full(b0.shape),
        full(W1.shape), full(b1.shape),
        full(W2.shape), full(b2.shape),
        full(Wl.shape), full(bl.shape),
    ]
    return pl.pallas_call(
        _mlp_body,
        grid=grid,
        in_specs=in_specs,
        out_specs=pl.BlockSpec((_BBM, 1), lambda i: (i, 0)),
        out_shape=jax.ShapeDtypeStruct((_B, 1), jnp.float32),
    )(pooled, W0, b0, W1, b1, W2, b2, Wl, bl)


def kernel(f0, f1, f2, f3, f4, f5, f6, f7,
           emb_f0, emb_f1, emb_f2, emb_f3, emb_f4, emb_f5, emb_f6, emb_f7,
           W0, b0, W1, b1, W2, b2, Wl, bl):
    idx_list = [
        f.astype(jnp.int32).reshape(_B * _L // 128, 128)
        for f in (f0, f1, f2, f3, f4, f5, f6, f7)
    ]
    tab = jnp.concatenate(
        [emb_f0, emb_f1, emb_f2, emb_f3, emb_f4, emb_f5, emb_f6, emb_f7]
    )
    scale = jnp.maximum(jnp.max(jnp.abs(tab)), 1e-30) / 32767.0
    tab_q = jnp.round(tab / scale).astype(jnp.int16)[:, jnp.asarray(_COL_PERM)]
    pooled = _sc_pool(idx_list, tab_q, jnp.full((16,), scale))
    return _tc_mlp(pooled, W0, b0, W1, b1, W2, b2, Wl, bl)


# speedup vs baseline: 1.0106x; 1.0106x over previous
import functools

import numpy as np

import jax
import jax.numpy as jnp
from jax import lax
from jax.experimental import pallas as pl
from jax.experimental.pallas import tpu as pltpu
from jax.experimental.pallas import tpu_sc as plsc

_NF = 8
_B = 16384
_L = 20
_V = 1024
_D = 64
_PARTS = 4
_ROWS_PER_W = _B // _PARTS
_CH = 32
_NSUB = _CH * _L // 128
_SCH = 256
_CPS = _SCH // _CH
_IDXROWS = _SCH * _L // 128
_BBM = 1024

_COL_PERM = np.empty(_D, np.int32)
for _k in range(_D // 32):
    for _j in range(16):
        _COL_PERM[_k * 32 + 2 * _j] = _k * 32 + _j
        _COL_PERM[_k * 32 + 2 * _j + 1] = _k * 32 + 16 + _j


def _sc_pool(idx_list, tab, scale):
    mesh = plsc.VectorSubcoreMesh(core_axis_name="c", subcore_axis_name="s")

    @functools.partial(
        pl.kernel,
        mesh=mesh,
        out_type=jax.ShapeDtypeStruct((_NF, _B, _D), jnp.float32),
        scratch_types=[
            pltpu.VMEM((_IDXROWS, 128), jnp.int32),
            pltpu.VMEM((_CH * _L, _D), jnp.int16),
            pltpu.VMEM((_CH * _L, _D), jnp.int16),
            pltpu.VMEM((_CH, _D), jnp.float32),
            pltpu.VMEM((_CH, _D), jnp.float32),
            pltpu.VMEM((16,), jnp.float32),
            pltpu.SemaphoreType.DMA,
            pltpu.SemaphoreType.DMA,
            pltpu.SemaphoreType.DMA,
            pltpu.SemaphoreType.DMA,
        ],
        compiler_params=pltpu.CompilerParams(
            needs_layout_passes=False, use_tc_tiling_on_sc=False
        ),
    )
    def pool(i0, i1, i2, i3, i4, i5, i6, i7, tab_hbm, scale_hbm, out_hbm,
             idx_s, rows0, rows1, outv0, outv1, scale_v,
             sg0, sg1, so0, so1):
        idx_refs = (i0, i1, i2, i3, i4, i5, i6, i7)
        wid = lax.axis_index("s") * 2 + lax.axis_index("c")
        fld = lax.shift_right_logical(wid, 2)
        part = lax.bitwise_and(wid, 3)
        rbase = part * _ROWS_PER_W
        pltpu.sync_copy(scale_hbm, scale_v)
        rows = (rows0, rows1)
        outv = (outv0, outv1)
        sg = (sg0, sg1)
        so = (so0, so1)

        def gathers(k, p):
            for j in range(_NSUB):
                pltpu.make_async_copy(
                    tab_hbm.at[idx_s.at[k * _NSUB + j]],
                    rows[p].at[pl.ds(j * 128, 128)],
                    sg[p],
                ).start()

        def drain_gathers(p):
            pltpu.make_async_copy(
                tab_hbm.at[pl.ds(0, _CH * _L)], rows[p], sg[p]
            ).wait()

        def out_copy(sbase, k, p):
            off = pl.multiple_of(sbase + k * _CH, _CH)
            return pltpu.make_async_copy(
                outv[p],
                out_hbm.at[fld, pl.ds(off, _CH)],
                so[p],
            )

        def compute(p):
            rv = rows[p]
            ov = outv[p]
            s = scale_v[...]

            def tree_sum(vals):
                while len(vals) > 1:
                    nxt = [
                        vals[2 * i] + vals[2 * i + 1]
                        for i in range(len(vals) // 2)
                    ]
                    if len(vals) % 2:
                        nxt.append(vals[-1])
                    vals = nxt
                return vals[0]

            def row_pool(r):
                g = r * _L
                for kk in range(_D // 32):
                    sl = pl.ds(kk * 32, 32)
                    evens = []
                    odds = []
                    for l in range(_L):
                        a, b = plsc.unpack(
                            rv[g + l, sl], format=plsc.PackFormat.INTERLEAVED
                        )
                        evens.append(a)
                        odds.append(b)
                    ov[r, pl.ds(kk * 32, 16)] = (
                        tree_sum(evens).astype(jnp.float32) * s
                    )
                    ov[r, pl.ds(kk * 32 + 16, 16)] = (
                        tree_sum(odds).astype(jnp.float32) * s
                    )

            def row_body(r2, carry):
                row_pool(r2 * 2)
                row_pool(r2 * 2 + 1)
                return carry

            lax.fori_loop(0, _CH // 2, row_body, 0)

        vbase = fld * _V

        def superchunk(si, carry):
            sbase = pl.multiple_of(rbase + si * _SCH, _SCH)
            idx_off = pl.multiple_of(sbase * _L // 128, _IDXROWS)
            for f in range(_NF):
                @pl.when(fld == f)
                def _():
                    pltpu.sync_copy(
                        idx_refs[f].at[pl.ds(idx_off, _IDXROWS)], idx_s
                    )

            def rebase_body(q, carry2):
                row = lax.shift_right_logical(q, 3)
                lane = lax.bitwise_and(q, 7) * 16
                sl = pl.ds(lane, 16)
                idx_s[row, sl] = idx_s[row, sl] + vbase
                return carry2

            lax.fori_loop(0, _IDXROWS * 8, rebase_body, 0)

            gathers(0, 0)
            for k in range(_CPS):
                p = k % 2
                if k + 1 < _CPS:
                    gathers(k + 1, 1 - p)
                drain_gathers(p)
                if k >= 2:
                    out_copy(sbase, k - 2, p).wait()
                compute(p)
                out_copy(sbase, k, p).start()
            out_copy(sbase, _CPS - 2, 0).wait()
            out_copy(sbase, _CPS - 1, 1).wait()
            return carry

        lax.fori_loop(0, _ROWS_PER_W // _SCH, superchunk, 0)

    return pool(*idx_list, tab, scale)


def _mlp_body(p_ref, w0_ref, b0_ref, w1_ref, b1_ref, w2_ref, b2_ref,
              wl_ref, bl_ref, out_ref):
    x = jnp.concatenate([p_ref[f] for f in range(_NF)], axis=-1)
    for w_ref, b_ref in ((w0_ref, b0_ref), (w1_ref, b1_ref), (w2_ref, b2_ref)):
        x = jnp.maximum(
            lax.dot(x, w_ref[...], preferred_element_type=jnp.float32)
            + b_ref[...][None, :],
            0.0,
        )
    out_ref[...] = (
        lax.dot(x, wl_ref[...], preferred_element_type=jnp.float32)
        + bl_ref[...][None, :]
    )


def _tc_mlp(pooled, W0, b0, W1, b1, W2, b2, Wl, bl):
    grid = (_B // _BBM,)
    full = lambda shape: pl.BlockSpec(shape, lambda i: tuple(0 for _ in shape))
    in_specs = [
        pl.BlockSpec((_NF, _BBM, _D), lambda i: (0, i, 0)),
        full(W0.shape), full(b0.shape),
        full(W1.shape), full(b1.shape),
        full(W2.shape), full(b2.shape),
        full(Wl.shape), full(bl.shape),
    ]
    return pl.pallas_call(
        _mlp_body,
        grid=grid,
        in_specs=in_specs,
        out_specs=pl.BlockSpec((_BBM, 1), lambda i: (i, 0)),
        out_shape=jax.ShapeDtypeStruct((_B, 1), jnp.float32),
    )(pooled, W0, b0, W1, b1, W2, b2, Wl, bl)


def kernel(f0, f1, f2, f3, f4, f5, f6, f7,
           emb_f0, emb_f1, emb_f2, emb_f3, emb_f4, emb_f5, emb_f6, emb_f7,
           W0, b0, W1, b1, W2, b2, Wl, bl):
    idx_list = [
        f.astype(jnp.int32).reshape(_B * _L // 128, 128)
        for f in (f0, f1, f2, f3, f4, f5, f6, f7)
    ]
    tab = jnp.concatenate(
        [emb_f0, emb_f1, emb_f2, emb_f3, emb_f4, emb_f5, emb_f6, emb_f7]
    )
    scale = jnp.maximum(jnp.max(jnp.abs(tab)), 1e-30) / 32767.0
    tab_q = jnp.round(tab / scale).astype(jnp.int16)[:, jnp.asarray(_COL_PERM)]
    pooled = _sc_pool(idx_list, tab_q, jnp.full((16,), scale))
    return _tc_mlp(pooled, W0, b0, W1, b1, W2, b2, Wl, bl)
